# baseline (speedup 1.0000x reference)
import functools

import jax
import jax.numpy as jnp
from jax import lax
from jax.experimental import pallas as pl
from jax.experimental.pallas import tpu as pltpu
from jax.experimental.pallas import tpu_sc as plsc

N = 10000
E = 320000
F_E = 16
N_GRAPHS = 16

NC = 2
NS = 16
NW = NC * NS
EDGES_PER_TILE = E // NW
BLK = 2000
NBLK = EDGES_PER_TILE // BLK
N_PAD = 10240
ROWS_PER_TILE = N_PAD // NS
CNT_ROWS = N_PAD // 16


def _sc_scatter_body(attr_hbm, col_hbm, sums_out, cnt_out,
                     attr_buf, idx_buf, cnt_t, sums_sh):
  c = lax.axis_index("c")
  s = lax.axis_index("s")
  wid = s * NC + c

  def init_row(i, _):
    attr_buf[i, :] = jnp.zeros((16,), jnp.float32)
    return 0
  lax.fori_loop(0, BLK, init_row, 0)

  def zero_cnt(i, _):
    cnt_t[i, :] = jnp.zeros((16,), jnp.float32)
    return 0
  lax.fori_loop(0, CNT_ROWS, zero_cnt, 0)

  row0 = s * ROWS_PER_TILE
  pltpu.sync_copy(attr_buf.at[pl.ds(0, ROWS_PER_TILE)],
                  sums_sh.at[pl.ds(row0, ROWS_PER_TILE)])
  plsc.subcore_barrier()

  base = wid * EDGES_PER_TILE
  ones16 = jnp.ones((16,), jnp.float32)

  def block(b, _):
    off = base + b * BLK
    pltpu.sync_copy(col_hbm.at[pl.ds(off, BLK)], idx_buf)
    pltpu.sync_copy(attr_hbm.at[pl.ds(off, BLK)], attr_buf)
    pltpu.sync_copy(attr_buf, sums_sh.at[idx_buf], add=True)

    def cnt_group(g, _):
      iv = idx_buf[pl.ds(g * 16, 16)]
      plsc.addupdate_scatter(
          cnt_t, [lax.shift_right_logical(iv, 4), lax.bitwise_and(iv, 15)],
          ones16)
      return 0
    lax.fori_loop(0, BLK // 16, cnt_group, 0)
    return 0
  lax.fori_loop(0, NBLK, block, 0)

  plsc.subcore_barrier()

  pltpu.sync_copy(sums_sh.at[pl.ds(row0, ROWS_PER_TILE)],
                  sums_out.at[c, pl.ds(row0, ROWS_PER_TILE)])
  pltpu.sync_copy(cnt_t, cnt_out.at[wid])


def _sc_scatter(edge_attr, col):
  mesh = plsc.VectorSubcoreMesh(core_axis_name="c", subcore_axis_name="s")
  kern = pl.kernel(
      _sc_scatter_body,
      out_type=[
          jax.ShapeDtypeStruct((NC, N_PAD, F_E), jnp.float32),
          jax.ShapeDtypeStruct((NW, CNT_ROWS, 16), jnp.float32),
      ],
      mesh=mesh,
      scratch_types=[
          pltpu.VMEM((BLK, F_E), jnp.float32),
          pltpu.VMEM((BLK,), jnp.int32),
          pltpu.VMEM((CNT_ROWS, 16), jnp.float32),
          pltpu.VMEM_SHARED((N_PAD, F_E), jnp.float32),
      ],
      compiler_params=pltpu.CompilerParams(use_tc_tiling_on_sc=False,
                                           needs_layout_passes=False),
  )
  return kern(edge_attr, col)


BN = 1000


def _mlp_body(x_ref, s0_ref, s1_ref, cnt_ref, batch_ref, u_ref,
              w1x_ref, w1e_ref, w1u_ref, b1_ref, w2_ref, b2_ref, out_ref):
  cnt = jnp.sum(cnt_ref[...], axis=1)
  e_agg = (s0_ref[...] + s1_ref[...]) / jnp.maximum(cnt, 1.0)[:, None]
  uw = jnp.dot(u_ref[...], w1u_ref[...], preferred_element_type=jnp.float32)
  b = batch_ref[0, 0, :]
  onehot = jnp.where(
      b[:, None] == lax.broadcasted_iota(jnp.int32, (1, N_GRAPHS), 1),
      1.0, 0.0)
  h = jnp.dot(x_ref[...], w1x_ref[...], preferred_element_type=jnp.float32)
  h += jnp.dot(e_agg, w1e_ref[...], preferred_element_type=jnp.float32)
  h += jnp.dot(onehot, uw, preferred_element_type=jnp.float32)
  h = jnp.maximum(h + b1_ref[...], 0.0)
  out_ref[...] = jnp.dot(h, w2_ref[...],
                         preferred_element_type=jnp.float32) + b2_ref[...]


def _mlp(x, s0, s1, cnt, batch3, u, w1x, w1e, w1u, b1, w2, b2):
  grid = N // BN
  full = lambda shape: pl.BlockSpec(shape, lambda i: (0,) * len(shape))
  return pl.pallas_call(
      _mlp_body,
      grid=(grid,),
      in_specs=[
          pl.BlockSpec((BN, 128), lambda i: (i, 0)),
          pl.BlockSpec((BN, F_E), lambda i: (i, 0)),
          pl.BlockSpec((BN, F_E), lambda i: (i, 0)),
          pl.BlockSpec((BN, NW), lambda i: (i, 0)),
          pl.BlockSpec((1, 1, BN), lambda i: (i, 0, 0)),
          full((N_GRAPHS, 128)),
          full((128, 128)),
          full((F_E, 128)),
          full((128, 128)),
          full((1, 128)),
          full((128, 128)),
          full((1, 128)),
      ],
      out_specs=pl.BlockSpec((BN, 128), lambda i: (i, 0)),
      out_shape=jax.ShapeDtypeStruct((N, 128), jnp.float32),
  )(x, s0, s1, cnt, batch3, u, w1x, w1e, w1u, b1, w2, b2)


@jax.jit
def kernel(x, edge_index, edge_attr, u, batch, W1, b1, W2, b2):
  col = edge_index[1].astype(jnp.int32)
  sums_p, cnt_p = _sc_scatter(edge_attr, col)
  cnt2 = cnt_p.reshape(NW, N_PAD)[:, :N].T
  batch3 = batch.astype(jnp.int32).reshape(N // BN, 1, BN)
  w1x = W1[:128]
  w1e = W1[128:128 + F_E]
  w1u = W1[128 + F_E:]
  return _mlp(x, sums_p[0, :N], sums_p[1, :N], cnt2, batch3, u,
              w1x, w1e, w1u, b1.reshape(1, 128), W2, b2.reshape(1, 128))


# speedup vs baseline: 1.1218x; 1.1218x over previous
import functools

import jax
import jax.numpy as jnp
from jax import lax
from jax.experimental import pallas as pl
from jax.experimental.pallas import tpu as pltpu
from jax.experimental.pallas import tpu_sc as plsc

N = 10000
E = 320000
F_E = 16
N_GRAPHS = 16

NC = 2
NS = 16
NW = NC * NS
EDGES_PER_TILE = E // NW
BLK = 1000
NBLK = EDGES_PER_TILE // BLK
N_PAD = 10240
ROWS_PER_TILE = N_PAD // NS


def _sc_scatter_body(ei_hbm, attr_hbm, sums_out, cnt_out,
                     attr0, attr1, idx0, idx1, ones_buf, sums_sh, cnt_sh,
                     isem0, isem1, jsem0, jsem1, ssem0, ssem1, csem0, csem1):
  c = lax.axis_index("c")
  s = lax.axis_index("s")
  wid = s * NC + c

  def init_row(i, _):
    attr0[i, :] = jnp.zeros((16,), jnp.float32)
    ones_buf[i, :] = jnp.ones((16,), jnp.float32)
    return 0
  lax.fori_loop(0, BLK, init_row, 0)

  row0 = s * ROWS_PER_TILE
  pltpu.sync_copy(attr0.at[pl.ds(0, ROWS_PER_TILE)],
                  sums_sh.at[pl.ds(row0, ROWS_PER_TILE)])
  pltpu.sync_copy(attr0.at[pl.ds(0, ROWS_PER_TILE)],
                  cnt_sh.at[pl.ds(row0, ROWS_PER_TILE)])
  plsc.subcore_barrier()

  base = wid * EDGES_PER_TILE
  attr = [attr0, attr1]
  idx = [idx0, idx1]
  isem = [isem0, isem1]
  jsem = [jsem0, jsem1]
  ssem = [ssem0, ssem1]
  csem = [csem0, csem1]

  def start_in(b, sl):
    off = base + b * BLK
    ha = pltpu.async_copy(attr_hbm.at[pl.ds(off, BLK)], attr[sl], isem[sl])
    hi = pltpu.async_copy(ei_hbm.at[1, pl.ds(off, BLK)], idx[sl], jsem[sl])
    return ha, hi

  in_pending = [None, None]
  sc_pending = [None, None]
  in_pending[0] = start_in(0, 0)

  for b in range(NBLK):
    sl = b & 1
    ha, hi = in_pending[sl]
    ha.wait()
    hi.wait()
    in_pending[sl] = None
    if b + 1 < NBLK:
      if sc_pending[1 - sl] is not None:
        hs, hc = sc_pending[1 - sl]
        hs.wait()
        hc.wait()
        sc_pending[1 - sl] = None
      in_pending[1 - sl] = start_in(b + 1, 1 - sl)
    hs = pltpu.async_copy(attr[sl], sums_sh.at[idx[sl]], ssem[sl], add=True)
    hc = pltpu.async_copy(ones_buf, cnt_sh.at[idx[sl]], csem[sl], add=True)
    sc_pending[sl] = (hs, hc)

  for sl in (0, 1):
    if sc_pending[sl] is not None:
      hs, hc = sc_pending[sl]
      hs.wait()
      hc.wait()

  plsc.subcore_barrier()

  pltpu.sync_copy(sums_sh.at[pl.ds(row0, ROWS_PER_TILE)],
                  sums_out.at[c, pl.ds(row0, ROWS_PER_TILE)])
  pltpu.sync_copy(cnt_sh.at[pl.ds(row0, ROWS_PER_TILE)],
                  cnt_out.at[c, pl.ds(row0, ROWS_PER_TILE)])


def _sc_scatter(edge_index, edge_attr):
  mesh = plsc.VectorSubcoreMesh(core_axis_name="c", subcore_axis_name="s")
  kern = pl.kernel(
      _sc_scatter_body,
      out_type=[
          jax.ShapeDtypeStruct((NC, N_PAD, F_E), jnp.float32),
          jax.ShapeDtypeStruct((NC, N_PAD, F_E), jnp.float32),
      ],
      mesh=mesh,
      scratch_types=[
          pltpu.VMEM((BLK, F_E), jnp.float32),
          pltpu.VMEM((BLK, F_E), jnp.float32),
          pltpu.VMEM((BLK,), jnp.int32),
          pltpu.VMEM((BLK,), jnp.int32),
          pltpu.VMEM((BLK, F_E), jnp.float32),
          pltpu.VMEM_SHARED((N_PAD, F_E), jnp.float32),
          pltpu.VMEM_SHARED((N_PAD, F_E), jnp.float32),
          pltpu.SemaphoreType.DMA,
          pltpu.SemaphoreType.DMA,
          pltpu.SemaphoreType.DMA,
          pltpu.SemaphoreType.DMA,
          pltpu.SemaphoreType.DMA,
          pltpu.SemaphoreType.DMA,
          pltpu.SemaphoreType.DMA,
          pltpu.SemaphoreType.DMA,
      ],
      compiler_params=pltpu.CompilerParams(use_tc_tiling_on_sc=False,
                                           needs_layout_passes=False),
  )
  return kern(edge_index, edge_attr)


BN = 640


def _mlp_body(x_ref, s_ref, c_ref, batch_ref, u_ref,
              w1x_ref, w1e_ref, w1u_ref, b1_ref, w2_ref, b2_ref, out_ref):
  cnt = c_ref[0] + c_ref[1]
  e_agg = (s_ref[0] + s_ref[1]) / jnp.maximum(cnt, 1.0)
  uw = jnp.dot(u_ref[...], w1u_ref[...], preferred_element_type=jnp.float32)
  b = batch_ref[0, 0, :]
  onehot = jnp.where(
      b[:, None] == lax.broadcasted_iota(jnp.int32, (1, N_GRAPHS), 1),
      1.0, 0.0)
  h = jnp.dot(x_ref[...], w1x_ref[...], preferred_element_type=jnp.float32)
  h += jnp.dot(e_agg, w1e_ref[...], preferred_element_type=jnp.float32)
  h += jnp.dot(onehot, uw, preferred_element_type=jnp.float32)
  h = jnp.maximum(h + b1_ref[...], 0.0)
  out_ref[...] = jnp.dot(h, w2_ref[...],
                         preferred_element_type=jnp.float32) + b2_ref[...]


def _mlp(x, sums_p, cnt_p, batch3, u, w1x, w1e, w1u, b1, w2, b2):
  grid = N_PAD // BN
  full = lambda shape: pl.BlockSpec(shape, lambda i: (0,) * len(shape))
  return pl.pallas_call(
      _mlp_body,
      grid=(grid,),
      in_specs=[
          pl.BlockSpec((BN, 128), lambda i: (i, 0)),
          pl.BlockSpec((NC, BN, F_E), lambda i: (0, i, 0)),
          pl.BlockSpec((NC, BN, F_E), lambda i: (0, i, 0)),
          pl.BlockSpec((1, 1, BN), lambda i: (i, 0, 0)),
          full((N_GRAPHS, 128)),
          full((128, 128)),
          full((F_E, 128)),
          full((128, 128)),
          full((1, 128)),
          full((128, 128)),
          full((1, 128)),
      ],
      out_specs=pl.BlockSpec((BN, 128), lambda i: (i, 0)),
      out_shape=jax.ShapeDtypeStruct((N, 128), jnp.float32),
  )(x, sums_p, cnt_p, batch3, u, w1x, w1e, w1u, b1, w2, b2)


@jax.jit
def kernel(x, edge_index, edge_attr, u, batch, W1, b1, W2, b2):
  sums_p, cnt_p = _sc_scatter(edge_index.astype(jnp.int32), edge_attr)
  batch_pad = jnp.concatenate(
      [batch.astype(jnp.int32), jnp.zeros((N_PAD - N,), jnp.int32)])
  batch3 = batch_pad.reshape(N_PAD // BN, 1, BN)
  w1x = W1[:128]
  w1e = W1[128:128 + F_E]
  w1u = W1[128 + F_E:]
  return _mlp(x, sums_p, cnt_p, batch3, u,
              w1x, w1e, w1u, b1.reshape(1, 128), W2, b2.reshape(1, 128))
